# baseline (speedup 1.0000x reference)
import jax
import jax.numpy as jnp
from jax import lax
from jax.experimental import pallas as pl
from jax.experimental.pallas import tpu as pltpu, tpu_sc as plsc

N = 10000
E = 320000
H = 128
FEAT = 128
ET = 4

NC = 2
NS = 16
NW = NC * NS
EPW = E // NW
CH = 80
NCHUNK = EPW // CH
NP = 10240
RPS = NP // NS

_F32 = jnp.float32


def _leaky(x):
    return jnp.where(x >= 0, x, 0.1 * x)


_R = 400


def _dot(a, b):
    return jnp.dot(a, b, preferred_element_type=_F32,
                   precision=lax.Precision.HIGHEST)


def _tc_prep0_body(x_ref, w1_ref, b1_ref, w2_ref, b2_ref, wa_ref, wb_ref,
                   t_ref, cf_ref, a_ref, bp_ref):
    h = _leaky(_leaky(_dot(x_ref[...], w1_ref[...]) + b1_ref[...]))
    cf = _leaky(_dot(h, w2_ref[...]) + b2_ref[...])
    cf_ref[...] = cf
    a_ref[...] = _dot(cf, wa_ref[...]).astype(jnp.bfloat16)
    b = _dot(cf, wb_ref[...])
    bp_ref[...] = (t_ref[...][:, None, :] + b[None, :, :]).astype(jnp.bfloat16)


def _tc_prep0(x, w1p, b1, w2, b2, wa, wb, t):
    row = pl.BlockSpec((_R, H), lambda i: (i, 0))
    wspec = pl.BlockSpec((H, H), lambda i: (0, 0))
    bspec = pl.BlockSpec((1, H), lambda i: (0, 0))
    tspec = pl.BlockSpec((ET, H), lambda i: (0, 0))
    bpspec = pl.BlockSpec((ET, _R, H), lambda i: (0, i, 0))
    return pl.pallas_call(
        _tc_prep0_body,
        grid=(N // _R,),
        in_specs=[row, wspec, bspec, wspec, bspec, wspec, wspec, tspec],
        out_specs=[row, row, bpspec],
        out_shape=[jax.ShapeDtypeStruct((N, H), _F32),
                   jax.ShapeDtypeStruct((N, H), jnp.bfloat16),
                   jax.ShapeDtypeStruct((ET, N, H), jnp.bfloat16)],
    )(x, w1p, b1, w2, b2, wa, wb, t)


def _tc_prep1_body(p_ref, wa_ref, wb_ref, t_ref, cf_ref, a_ref, bp_ref):
    cf = p_ref[0] + p_ref[1]
    cf_ref[...] = cf
    a_ref[...] = _dot(cf, wa_ref[...]).astype(jnp.bfloat16)
    b = _dot(cf, wb_ref[...])
    bp_ref[...] = (t_ref[...][:, None, :] + b[None, :, :]).astype(jnp.bfloat16)


def _tc_prep1(p, wa, wb, t):
    row = pl.BlockSpec((_R, H), lambda i: (i, 0))
    pspec = pl.BlockSpec((2, _R, H), lambda i: (0, i, 0))
    wspec = pl.BlockSpec((H, H), lambda i: (0, 0))
    tspec = pl.BlockSpec((ET, H), lambda i: (0, 0))
    bpspec = pl.BlockSpec((ET, _R, H), lambda i: (0, i, 0))
    return pl.pallas_call(
        _tc_prep1_body,
        grid=(N // _R,),
        in_specs=[pspec, wspec, wspec, tspec],
        out_specs=[row, row, bpspec],
        out_shape=[jax.ShapeDtypeStruct((N, H), _F32),
                   jax.ShapeDtypeStruct((N, H), jnp.bfloat16),
                   jax.ShapeDtypeStruct((ET, N, H), jnp.bfloat16)],
    )(p, wa, wb, t)


def _tc_final_body(cf0_ref, cf1_ref, p_ref, w0_ref, w1_ref, w2_ref, bs_ref,
                   out_ref):
    cf2 = p_ref[0] + p_ref[1]
    acc = _dot(cf0_ref[...], w0_ref[...])
    acc += _dot(cf1_ref[...], w1_ref[...])
    acc += _dot(cf2, w2_ref[...])
    out_ref[...] = _leaky(acc + bs_ref[...])


def _tc_final(cf0, cf1, p, ws0, ws1, ws2, bs):
    row = pl.BlockSpec((_R, H), lambda i: (i, 0))
    pspec = pl.BlockSpec((2, _R, H), lambda i: (0, i, 0))
    wspec = pl.BlockSpec((H, FEAT), lambda i: (0, 0))
    bspec = pl.BlockSpec((1, FEAT), lambda i: (0, 0))
    return pl.pallas_call(
        _tc_final_body,
        grid=(N // _R,),
        in_specs=[row, row, pspec, wspec, wspec, wspec, bspec],
        out_specs=row,
        out_shape=jax.ShapeDtypeStruct((N, FEAT), _F32),
    )(cf0, cf1, p, ws0, ws1, ws2, bs)


def _sc_iter_body(a_hbm, bp_hbm, ii_hbm, out_hbm,
                  acc, idx0, idx1, bufa0, bufb0, bufa1, bufb1, msg,
                  semg0, semg1):
    c = lax.axis_index("c")
    s = lax.axis_index("s")
    wid = s * NC + c

    def zero_body(e, _):
        for j in range(H // 16):
            msg[e, pl.ds(j * 16, 16)] = jnp.zeros((16,), _F32)
        return 0
    lax.fori_loop(0, CH, zero_body, 0)
    for r in range(RPS // CH):
        pltpu.sync_copy(msg.at[pl.ds(0, CH)],
                        acc.at[pl.ds(s * RPS + r * CH, CH)])
    plsc.subcore_barrier()

    bufs = ((bufa0, bufb0, idx0, semg0), (bufa1, bufb1, idx1, semg1))

    def issue(k, p):
        ba, bb, ix, sg = bufs[p]
        pltpu.sync_copy(ii_hbm.at[wid, k], ix)
        pltpu.async_copy(a_hbm.at[ix.at[0]], ba, sg)
        pltpu.async_copy(bp_hbm.at[ix.at[1]], bb, sg)

    def process(k, p, pre):
        ba, bb, ix, sg = bufs[p]
        pltpu.make_async_copy(a_hbm.at[ix.at[0]], ba, sg).wait()
        pltpu.make_async_copy(bp_hbm.at[ix.at[1]], bb, sg).wait()

        zero = jnp.zeros((16,), _F32)
        himask = jnp.full((16,), -65536, jnp.int32)

        def edge_conv(e, _):
            for j in range(H // 32):
                va = ba[e, pl.ds(j * 16, 16)]
                vb = bb[e, pl.ds(j * 16, 16)]
                a_lo = plsc.bitcast(va << 16, _F32)
                b_lo = plsc.bitcast(vb << 16, _F32)
                a_hi = plsc.bitcast(va & himask, _F32)
                b_hi = plsc.bitcast(vb & himask, _F32)
                msg[e, pl.ds(j * 32, 16)] = jnp.maximum(a_lo + b_lo, zero)
                msg[e, pl.ds(j * 32 + 16, 16)] = jnp.maximum(a_hi + b_hi,
                                                             zero)
            return 0
        lax.fori_loop(0, CH, edge_conv, 0)
        pltpu.sync_copy(msg, acc.at[ix.at[0]], add=True)
        if pre:
            issue(k + 2, p)

    issue(0, 0)
    issue(1, 1)

    def pair_body(i, _):
        k0 = 2 * i
        process(k0, 0, True)
        process(k0 + 1, 1, True)
        return 0
    lax.fori_loop(0, (NCHUNK - 3) // 2, pair_body, 0)
    process(NCHUNK - 3, 0, True)
    process(NCHUNK - 2, 1, False)
    process(NCHUNK - 1, 0, False)

    plsc.subcore_barrier()

    pltpu.sync_copy(acc.at[pl.ds(s * RPS, RPS)],
                    out_hbm.at[c, pl.ds(s * RPS, RPS)])


_sc_iter = pl.kernel(
    _sc_iter_body,
    out_type=jax.ShapeDtypeStruct((NC, NP, H), _F32),
    mesh=plsc.VectorSubcoreMesh(core_axis_name="c", subcore_axis_name="s"),
    compiler_params=pltpu.CompilerParams(use_tc_tiling_on_sc=False,
                                         needs_layout_passes=False),
    scratch_types=[
        pltpu.VMEM_SHARED((NP, H), _F32),
        pltpu.VMEM((2, CH), jnp.int32),
        pltpu.VMEM((2, CH), jnp.int32),
        pltpu.VMEM((CH, H // 2), jnp.int32),
        pltpu.VMEM((CH, H // 2), jnp.int32),
        pltpu.VMEM((CH, H // 2), jnp.int32),
        pltpu.VMEM((CH, H // 2), jnp.int32),
        pltpu.VMEM((CH, H), _F32),
        pltpu.SemaphoreType.DMA,
        pltpu.SemaphoreType.DMA,
    ],
)


def kernel(child_feats, edge_indices, edge_type_onehot, lengths,
           W1, b1, W2, b2, We0, be0, We1, be1, Ws, bs):
    x = child_feats[0]
    ef = edge_indices[0, :, 0].astype(jnp.int32)
    et = edge_indices[0, :, 1].astype(jnp.int32)
    ty = jnp.argmax(edge_type_onehot[0], axis=1).astype(jnp.int32)
    ebt = ty * N + et

    ii = jnp.stack([ef.reshape(NW, NCHUNK, CH),
                    ebt.reshape(NW, NCHUNK, CH)], axis=2)

    w1p = jnp.zeros((FEAT, H), _F32).at[:W1.shape[0]].set(W1)

    g = jnp.arange(H) // 32 * 32
    m = jnp.arange(H) % 32
    perm = g + jnp.where(m % 2 == 0, m // 2, 16 + m // 2)

    t0 = (We0[2 * H:] + be0)[:, perm]
    t1 = (We1[2 * H:] + be1)[:, perm]

    def _pack(tbl):
        m = tbl.shape[0]
        return lax.bitcast_convert_type(tbl.reshape(m, H // 2, 2), jnp.int32)

    cf0, a0, bp0 = _tc_prep0(x, w1p, b1.reshape(1, H), W2, b2.reshape(1, H),
                             We0[:H][:, perm], We0[H:2 * H][:, perm], t0)
    p0 = _sc_iter(_pack(a0), _pack(bp0.reshape(ET * N, H)), ii)
    cf1, a1, bp1 = _tc_prep1(p0, We1[:H][:, perm], We1[H:2 * H][:, perm], t1)
    p1 = _sc_iter(_pack(a1), _pack(bp1.reshape(ET * N, H)), ii)
    return _tc_final(cf0, cf1, p1, Ws[:H], Ws[H:2 * H], Ws[2 * H:],
                     bs.reshape(1, FEAT))


# speedup vs baseline: 2.6354x; 2.6354x over previous
import jax
import jax.numpy as jnp
from jax import lax
from jax.experimental import pallas as pl
from jax.experimental.pallas import tpu as pltpu, tpu_sc as plsc

N = 10000
E = 320000
H = 128
FEAT = 128
ET = 4

NC = 2
NS = 16
NW = NC * NS
EPW = E // NW
CH = 80
NCHUNK = EPW // CH
NP = 10240
RPS = NP // NS

_F32 = jnp.float32


def _leaky(x):
    return jnp.where(x >= 0, x, 0.1 * x)


_R = 400


def _dot(a, b):
    return jnp.dot(a, b, preferred_element_type=_F32,
                   precision=lax.Precision.HIGHEST)


def _tc_prep0_body(x_ref, w1_ref, b1_ref, w2_ref, b2_ref, wa_ref, wb_ref,
                   t_ref, cf_ref, a_ref, bp_ref):
    h = _leaky(_leaky(_dot(x_ref[...], w1_ref[...]) + b1_ref[...]))
    cf = _leaky(_dot(h, w2_ref[...]) + b2_ref[...])
    cf_ref[...] = cf
    a_ref[...] = _dot(cf, wa_ref[...])
    b = _dot(cf, wb_ref[...])
    bp_ref[...] = t_ref[...][:, None, :] + b[None, :, :]


def _tc_prep0(x, w1p, b1, w2, b2, wa, wb, t):
    row = pl.BlockSpec((_R, H), lambda i: (i, 0))
    wspec = pl.BlockSpec((H, H), lambda i: (0, 0))
    bspec = pl.BlockSpec((1, H), lambda i: (0, 0))
    tspec = pl.BlockSpec((ET, H), lambda i: (0, 0))
    bpspec = pl.BlockSpec((ET, _R, H), lambda i: (0, i, 0))
    return pl.pallas_call(
        _tc_prep0_body,
        grid=(N // _R,),
        in_specs=[row, wspec, bspec, wspec, bspec, wspec, wspec, tspec],
        out_specs=[row, row, bpspec],
        out_shape=[jax.ShapeDtypeStruct((N, H), _F32),
                   jax.ShapeDtypeStruct((N, H), _F32),
                   jax.ShapeDtypeStruct((ET, N, H), _F32)],
    )(x, w1p, b1, w2, b2, wa, wb, t)


def _tc_prep1_body(p_ref, wa_ref, wb_ref, t_ref, cf_ref, a_ref, bp_ref):
    cf = p_ref[0] + p_ref[1]
    cf_ref[...] = cf
    a_ref[...] = _dot(cf, wa_ref[...])
    b = _dot(cf, wb_ref[...])
    bp_ref[...] = t_ref[...][:, None, :] + b[None, :, :]


def _tc_prep1(p, wa, wb, t):
    row = pl.BlockSpec((_R, H), lambda i: (i, 0))
    pspec = pl.BlockSpec((2, _R, H), lambda i: (0, i, 0))
    wspec = pl.BlockSpec((H, H), lambda i: (0, 0))
    tspec = pl.BlockSpec((ET, H), lambda i: (0, 0))
    bpspec = pl.BlockSpec((ET, _R, H), lambda i: (0, i, 0))
    return pl.pallas_call(
        _tc_prep1_body,
        grid=(N // _R,),
        in_specs=[pspec, wspec, wspec, tspec],
        out_specs=[row, row, bpspec],
        out_shape=[jax.ShapeDtypeStruct((N, H), _F32),
                   jax.ShapeDtypeStruct((N, H), _F32),
                   jax.ShapeDtypeStruct((ET, N, H), _F32)],
    )(p, wa, wb, t)


def _tc_final_body(cf0_ref, cf1_ref, p_ref, w0_ref, w1_ref, w2_ref, bs_ref,
                   out_ref):
    cf2 = p_ref[0] + p_ref[1]
    acc = _dot(cf0_ref[...], w0_ref[...])
    acc += _dot(cf1_ref[...], w1_ref[...])
    acc += _dot(cf2, w2_ref[...])
    out_ref[...] = _leaky(acc + bs_ref[...])


def _tc_final(cf0, cf1, p, ws0, ws1, ws2, bs):
    row = pl.BlockSpec((_R, H), lambda i: (i, 0))
    pspec = pl.BlockSpec((2, _R, H), lambda i: (0, i, 0))
    wspec = pl.BlockSpec((H, FEAT), lambda i: (0, 0))
    bspec = pl.BlockSpec((1, FEAT), lambda i: (0, 0))
    return pl.pallas_call(
        _tc_final_body,
        grid=(N // _R,),
        in_specs=[row, row, pspec, wspec, wspec, wspec, bspec],
        out_specs=row,
        out_shape=jax.ShapeDtypeStruct((N, FEAT), _F32),
    )(cf0, cf1, p, ws0, ws1, ws2, bs)


def _sc_iter_body(a_hbm, bp_hbm, ii_hbm, out_hbm,
                  acc, ring, bufa0, bufb0, bufa1, bufb1,
                  semg0, semg1, semi0, semi1):
    c = lax.axis_index("c")
    s = lax.axis_index("s")
    wid = s * NC + c

    def zero_body(e, _):
        for j in range(H // 16):
            bufa0[e, pl.ds(j * 16, 16)] = jnp.zeros((16,), _F32)
        return 0
    lax.fori_loop(0, CH, zero_body, 0)
    for r in range(RPS // CH):
        pltpu.sync_copy(bufa0.at[pl.ds(0, CH)],
                        acc.at[pl.ds(s * RPS + r * CH, CH)])
    plsc.subcore_barrier()

    bufs = ((bufa0, bufb0, semg0, semi0), (bufa1, bufb1, semg1, semi1))

    def idx_load(k2, sem):
        pltpu.async_copy(ii_hbm.at[wid, k2], ring.at[lax.rem(k2, 4)], sem)

    def issue(k, p):
        ba, bb, sg, _ = bufs[p]
        m = lax.rem(k, 4)
        pltpu.async_copy(a_hbm.at[ring.at[m, 0]], ba, sg)
        pltpu.async_copy(bp_hbm.at[ring.at[m, 1]], bb, sg)

    def process(k, p, pre, pre_idx):
        ba, bb, sg, si = bufs[p]
        m = lax.rem(k, 4)
        pltpu.make_async_copy(a_hbm.at[ring.at[m, 0]], ba, sg).wait()
        pltpu.make_async_copy(bp_hbm.at[ring.at[m, 1]], bb, sg).wait()

        def edge_body(e, _):
            for j in range(H // 16):
                sl = pl.ds(j * 16, 16)
                ba[e, sl] = jnp.maximum(ba[e, sl] + bb[e, sl], 0.0)
            return 0
        lax.fori_loop(0, CH, edge_body, 0)
        pltpu.sync_copy(ba, acc.at[ring.at[m, 0]], add=True)
        if pre:
            pltpu.make_async_copy(ii_hbm.at[wid, k + 2],
                                  ring.at[lax.rem(k + 2, 4)], si).wait()
            issue(k + 2, p)
        if pre_idx:
            idx_load(k + 4, si)

    pltpu.sync_copy(ii_hbm.at[wid, 0], ring.at[0])
    pltpu.sync_copy(ii_hbm.at[wid, 1], ring.at[1])
    issue(0, 0)
    issue(1, 1)
    idx_load(2, semi0)
    idx_load(3, semi1)

    def pair_body(i, _):
        k0 = 2 * i
        process(k0, 0, True, True)
        process(k0 + 1, 1, True, True)
        return 0
    lax.fori_loop(0, (NCHUNK - 5) // 2, pair_body, 0)
    process(NCHUNK - 5, 0, True, True)
    process(NCHUNK - 4, 1, True, False)
    process(NCHUNK - 3, 0, True, False)
    process(NCHUNK - 2, 1, False, False)
    process(NCHUNK - 1, 0, False, False)

    plsc.subcore_barrier()

    pltpu.sync_copy(acc.at[pl.ds(s * RPS, RPS)],
                    out_hbm.at[c, pl.ds(s * RPS, RPS)])


_sc_iter = pl.kernel(
    _sc_iter_body,
    out_type=jax.ShapeDtypeStruct((NC, NP, H), _F32),
    mesh=plsc.VectorSubcoreMesh(core_axis_name="c", subcore_axis_name="s"),
    scratch_types=[
        pltpu.VMEM_SHARED((NP, H), _F32),
        pltpu.VMEM((4, 2, CH), jnp.int32),
        pltpu.VMEM((CH, H), _F32),
        pltpu.VMEM((CH, H), _F32),
        pltpu.VMEM((CH, H), _F32),
        pltpu.VMEM((CH, H), _F32),
        pltpu.SemaphoreType.DMA,
        pltpu.SemaphoreType.DMA,
        pltpu.SemaphoreType.DMA,
        pltpu.SemaphoreType.DMA,
    ],
)


def kernel(child_feats, edge_indices, edge_type_onehot, lengths,
           W1, b1, W2, b2, We0, be0, We1, be1, Ws, bs):
    x = child_feats[0]
    ef = edge_indices[0, :, 0].astype(jnp.int32)
    et = edge_indices[0, :, 1].astype(jnp.int32)
    ty = jnp.argmax(edge_type_onehot[0], axis=1).astype(jnp.int32)
    ebt = ty * N + et

    ii = jnp.stack([ef.reshape(NW, NCHUNK, CH),
                    ebt.reshape(NW, NCHUNK, CH)], axis=2)

    w1p = jnp.zeros((FEAT, H), _F32).at[:W1.shape[0]].set(W1)
    t0 = We0[2 * H:] + be0
    t1 = We1[2 * H:] + be1

    cf0, a0, bp0 = _tc_prep0(x, w1p, b1.reshape(1, H), W2, b2.reshape(1, H),
                             We0[:H], We0[H:2 * H], t0)
    p0 = _sc_iter(a0, bp0.reshape(ET * N, H), ii)
    cf1, a1, bp1 = _tc_prep1(p0, We1[:H], We1[H:2 * H], t1)
    p1 = _sc_iter(a1, bp1.reshape(ET * N, H), ii)
    return _tc_final(cf0, cf1, p1, Ws[:H], Ws[H:2 * H], Ws[2 * H:],
                     bs.reshape(1, FEAT))


# speedup vs baseline: 2.7206x; 1.0323x over previous
import jax
import jax.numpy as jnp
from jax import lax
from jax.experimental import pallas as pl
from jax.experimental.pallas import tpu as pltpu, tpu_sc as plsc

N = 10000
E = 320000
H = 128
FEAT = 128
ET = 4

NC = 2
NS = 16
NW = NC * NS
EPW = E // NW
CH = 80
NCHUNK = EPW // CH
NP = 10240
RPS = NP // NS

_F32 = jnp.float32


def _leaky(x):
    return jnp.where(x >= 0, x, 0.1 * x)


_R = 400


def _dot(a, b):
    return jnp.dot(a, b, preferred_element_type=_F32)


def _tc_prep0_body(x_ref, w1_ref, b1_ref, w2_ref, b2_ref, wa_ref, wb_ref,
                   t_ref, cf_ref, a_ref, bp_ref):
    h = _leaky(_leaky(_dot(x_ref[...], w1_ref[...]) + b1_ref[...]))
    cf = _leaky(_dot(h, w2_ref[...]) + b2_ref[...])
    cf_ref[...] = cf
    a_ref[...] = _dot(cf, wa_ref[...])
    b = _dot(cf, wb_ref[...])
    bp_ref[...] = t_ref[...][:, None, :] + b[None, :, :]


def _tc_prep0(x, w1p, b1, w2, b2, wa, wb, t):
    row = pl.BlockSpec((_R, H), lambda i: (i, 0))
    wspec = pl.BlockSpec((H, H), lambda i: (0, 0))
    bspec = pl.BlockSpec((1, H), lambda i: (0, 0))
    tspec = pl.BlockSpec((ET, H), lambda i: (0, 0))
    bpspec = pl.BlockSpec((ET, _R, H), lambda i: (0, i, 0))
    return pl.pallas_call(
        _tc_prep0_body,
        grid=(N // _R,),
        in_specs=[row, wspec, bspec, wspec, bspec, wspec, wspec, tspec],
        out_specs=[row, row, bpspec],
        out_shape=[jax.ShapeDtypeStruct((N, H), _F32),
                   jax.ShapeDtypeStruct((N, H), _F32),
                   jax.ShapeDtypeStruct((ET, N, H), _F32)],
    )(x, w1p, b1, w2, b2, wa, wb, t)


def _tc_prep1_body(p_ref, wa_ref, wb_ref, t_ref, cf_ref, a_ref, bp_ref):
    cf = p_ref[0] + p_ref[1]
    cf_ref[...] = cf
    a_ref[...] = _dot(cf, wa_ref[...])
    b = _dot(cf, wb_ref[...])
    bp_ref[...] = t_ref[...][:, None, :] + b[None, :, :]


def _tc_prep1(p, wa, wb, t):
    row = pl.BlockSpec((_R, H), lambda i: (i, 0))
    pspec = pl.BlockSpec((2, _R, H), lambda i: (0, i, 0))
    wspec = pl.BlockSpec((H, H), lambda i: (0, 0))
    tspec = pl.BlockSpec((ET, H), lambda i: (0, 0))
    bpspec = pl.BlockSpec((ET, _R, H), lambda i: (0, i, 0))
    return pl.pallas_call(
        _tc_prep1_body,
        grid=(N // _R,),
        in_specs=[pspec, wspec, wspec, tspec],
        out_specs=[row, row, bpspec],
        out_shape=[jax.ShapeDtypeStruct((N, H), _F32),
                   jax.ShapeDtypeStruct((N, H), _F32),
                   jax.ShapeDtypeStruct((ET, N, H), _F32)],
    )(p, wa, wb, t)


def _tc_final_body(cf0_ref, cf1_ref, p_ref, w0_ref, w1_ref, w2_ref, bs_ref,
                   out_ref):
    cf2 = p_ref[0] + p_ref[1]
    acc = _dot(cf0_ref[...], w0_ref[...])
    acc += _dot(cf1_ref[...], w1_ref[...])
    acc += _dot(cf2, w2_ref[...])
    out_ref[...] = _leaky(acc + bs_ref[...])


def _tc_final(cf0, cf1, p, ws0, ws1, ws2, bs):
    row = pl.BlockSpec((_R, H), lambda i: (i, 0))
    pspec = pl.BlockSpec((2, _R, H), lambda i: (0, i, 0))
    wspec = pl.BlockSpec((H, FEAT), lambda i: (0, 0))
    bspec = pl.BlockSpec((1, FEAT), lambda i: (0, 0))
    return pl.pallas_call(
        _tc_final_body,
        grid=(N // _R,),
        in_specs=[row, row, pspec, wspec, wspec, wspec, bspec],
        out_specs=row,
        out_shape=jax.ShapeDtypeStruct((N, FEAT), _F32),
    )(cf0, cf1, p, ws0, ws1, ws2, bs)


def _sc_iter_body(a_hbm, bp_hbm, ii_hbm, out_hbm,
                  acc, ring, bufa0, bufb0, bufa1, bufb1,
                  semg0, semg1, semi0, semi1):
    c = lax.axis_index("c")
    s = lax.axis_index("s")
    wid = s * NC + c

    def zero_body(e, _):
        for j in range(H // 16):
            bufa0[e, pl.ds(j * 16, 16)] = jnp.zeros((16,), _F32)
        return 0
    lax.fori_loop(0, CH, zero_body, 0)
    for r in range(RPS // CH):
        pltpu.sync_copy(bufa0.at[pl.ds(0, CH)],
                        acc.at[pl.ds(s * RPS + r * CH, CH)])
    plsc.subcore_barrier()

    bufs = ((bufa0, bufb0, semg0, semi0), (bufa1, bufb1, semg1, semi1))

    def idx_load(k2, sem):
        pltpu.async_copy(ii_hbm.at[wid, k2], ring.at[lax.rem(k2, 4)], sem)

    def issue(k, p):
        ba, bb, sg, _ = bufs[p]
        m = lax.rem(k, 4)
        pltpu.async_copy(a_hbm.at[ring.at[m, 0]], ba, sg)
        pltpu.async_copy(bp_hbm.at[ring.at[m, 1]], bb, sg)

    def process(k, p, pre, pre_idx):
        ba, bb, sg, si = bufs[p]
        m = lax.rem(k, 4)
        pltpu.make_async_copy(a_hbm.at[ring.at[m, 0]], ba, sg).wait()
        pltpu.make_async_copy(bp_hbm.at[ring.at[m, 1]], bb, sg).wait()

        def edge_body(e, _):
            for j in range(H // 16):
                sl = pl.ds(j * 16, 16)
                ba[e, sl] = jnp.maximum(ba[e, sl] + bb[e, sl], 0.0)
            return 0
        lax.fori_loop(0, CH, edge_body, 0)
        pltpu.sync_copy(ba, acc.at[ring.at[m, 0]], add=True)
        if pre:
            pltpu.make_async_copy(ii_hbm.at[wid, k + 2],
                                  ring.at[lax.rem(k + 2, 4)], si).wait()
            issue(k + 2, p)
        if pre_idx:
            idx_load(k + 4, si)

    pltpu.sync_copy(ii_hbm.at[wid, 0], ring.at[0])
    pltpu.sync_copy(ii_hbm.at[wid, 1], ring.at[1])
    issue(0, 0)
    issue(1, 1)
    idx_load(2, semi0)
    idx_load(3, semi1)

    def pair_body(i, _):
        k0 = 2 * i
        process(k0, 0, True, True)
        process(k0 + 1, 1, True, True)
        return 0
    lax.fori_loop(0, (NCHUNK - 5) // 2, pair_body, 0)
    process(NCHUNK - 5, 0, True, True)
    process(NCHUNK - 4, 1, True, False)
    process(NCHUNK - 3, 0, True, False)
    process(NCHUNK - 2, 1, False, False)
    process(NCHUNK - 1, 0, False, False)

    plsc.subcore_barrier()

    pltpu.sync_copy(acc.at[pl.ds(s * RPS, RPS)],
                    out_hbm.at[c, pl.ds(s * RPS, RPS)])


_sc_iter = pl.kernel(
    _sc_iter_body,
    out_type=jax.ShapeDtypeStruct((NC, NP, H), _F32),
    mesh=plsc.VectorSubcoreMesh(core_axis_name="c", subcore_axis_name="s"),
    scratch_types=[
        pltpu.VMEM_SHARED((NP, H), _F32),
        pltpu.VMEM((4, 2, CH), jnp.int32),
        pltpu.VMEM((CH, H), _F32),
        pltpu.VMEM((CH, H), _F32),
        pltpu.VMEM((CH, H), _F32),
        pltpu.VMEM((CH, H), _F32),
        pltpu.SemaphoreType.DMA,
        pltpu.SemaphoreType.DMA,
        pltpu.SemaphoreType.DMA,
        pltpu.SemaphoreType.DMA,
    ],
)


def kernel(child_feats, edge_indices, edge_type_onehot, lengths,
           W1, b1, W2, b2, We0, be0, We1, be1, Ws, bs):
    x = child_feats[0]
    ef = edge_indices[0, :, 0].astype(jnp.int32)
    et = edge_indices[0, :, 1].astype(jnp.int32)
    ty = jnp.argmax(edge_type_onehot[0], axis=1).astype(jnp.int32)
    ebt = ty * N + et

    ii = jnp.stack([ef.reshape(NW, NCHUNK, CH),
                    ebt.reshape(NW, NCHUNK, CH)], axis=2)

    w1p = jnp.zeros((FEAT, H), _F32).at[:W1.shape[0]].set(W1)
    t0 = We0[2 * H:] + be0
    t1 = We1[2 * H:] + be1

    cf0, a0, bp0 = _tc_prep0(x, w1p, b1.reshape(1, H), W2, b2.reshape(1, H),
                             We0[:H], We0[H:2 * H], t0)
    p0 = _sc_iter(a0, bp0.reshape(ET * N, H), ii)
    cf1, a1, bp1 = _tc_prep1(p0, We1[:H], We1[H:2 * H], t1)
    p1 = _sc_iter(a1, bp1.reshape(ET * N, H), ii)
    return _tc_final(cf0, cf1, p1, Ws[:H], Ws[H:2 * H], Ws[2 * H:],
                     bs.reshape(1, FEAT))


# speedup vs baseline: 2.7401x; 1.0072x over previous
import jax
import jax.numpy as jnp
from jax import lax
from jax.experimental import pallas as pl
from jax.experimental.pallas import tpu as pltpu, tpu_sc as plsc

N = 10000
E = 320000
H = 128
FEAT = 128
ET = 4

NC = 2
NS = 16
NW = NC * NS
EPW = E // NW
CH = 80
NCHUNK = EPW // CH
NP = 10240
RPS = NP // NS

_F32 = jnp.float32


def _leaky(x):
    return jnp.where(x >= 0, x, 0.1 * x)


_R = 400


def _dot(a, b):
    return jnp.dot(a, b, preferred_element_type=_F32)


def _tc_prep0_body(x_ref, w1_ref, b1_ref, w2_ref, b2_ref, wa_ref, wb_ref,
                   t_ref, cf_ref, a_ref, bp_ref):
    h = _leaky(_leaky(_dot(x_ref[...], w1_ref[...]) + b1_ref[...]))
    cf = _leaky(_dot(h, w2_ref[...]) + b2_ref[...])
    cf_ref[...] = cf
    a_ref[...] = _dot(cf, wa_ref[...])
    b = _dot(cf, wb_ref[...])
    bp_ref[...] = t_ref[...][:, None, :] + b[None, :, :]


def _tc_prep0(x, w1p, b1, w2, b2, wa, wb, t):
    row = pl.BlockSpec((_R, H), lambda i: (i, 0))
    wspec = pl.BlockSpec((H, H), lambda i: (0, 0))
    bspec = pl.BlockSpec((1, H), lambda i: (0, 0))
    tspec = pl.BlockSpec((ET, H), lambda i: (0, 0))
    bpspec = pl.BlockSpec((ET, _R, H), lambda i: (0, i, 0))
    return pl.pallas_call(
        _tc_prep0_body,
        grid=(N // _R,),
        in_specs=[row, wspec, bspec, wspec, bspec, wspec, wspec, tspec],
        out_specs=[row, row, bpspec],
        out_shape=[jax.ShapeDtypeStruct((N, H), _F32),
                   jax.ShapeDtypeStruct((N, H), _F32),
                   jax.ShapeDtypeStruct((ET, N, H), _F32)],
    )(x, w1p, b1, w2, b2, wa, wb, t)


def _tc_prep1_body(p_ref, wa_ref, wb_ref, t_ref, cf_ref, a_ref, bp_ref):
    cf = p_ref[0] + p_ref[1]
    cf_ref[...] = cf
    a_ref[...] = _dot(cf, wa_ref[...])
    b = _dot(cf, wb_ref[...])
    bp_ref[...] = t_ref[...][:, None, :] + b[None, :, :]


def _tc_prep1(p, wa, wb, t):
    row = pl.BlockSpec((_R, H), lambda i: (i, 0))
    pspec = pl.BlockSpec((2, _R, H), lambda i: (0, i, 0))
    wspec = pl.BlockSpec((H, H), lambda i: (0, 0))
    tspec = pl.BlockSpec((ET, H), lambda i: (0, 0))
    bpspec = pl.BlockSpec((ET, _R, H), lambda i: (0, i, 0))
    return pl.pallas_call(
        _tc_prep1_body,
        grid=(N // _R,),
        in_specs=[pspec, wspec, wspec, tspec],
        out_specs=[row, row, bpspec],
        out_shape=[jax.ShapeDtypeStruct((N, H), _F32),
                   jax.ShapeDtypeStruct((N, H), _F32),
                   jax.ShapeDtypeStruct((ET, N, H), _F32)],
    )(p, wa, wb, t)


def _tc_final_body(cf0_ref, cf1_ref, p_ref, w0_ref, w1_ref, w2_ref, bs_ref,
                   out_ref):
    cf2 = p_ref[0] + p_ref[1]
    acc = _dot(cf0_ref[...], w0_ref[...])
    acc += _dot(cf1_ref[...], w1_ref[...])
    acc += _dot(cf2, w2_ref[...])
    out_ref[...] = _leaky(acc + bs_ref[...])


def _tc_final(cf0, cf1, p, ws0, ws1, ws2, bs):
    row = pl.BlockSpec((_R, H), lambda i: (i, 0))
    pspec = pl.BlockSpec((2, _R, H), lambda i: (0, i, 0))
    wspec = pl.BlockSpec((H, FEAT), lambda i: (0, 0))
    bspec = pl.BlockSpec((1, FEAT), lambda i: (0, 0))
    return pl.pallas_call(
        _tc_final_body,
        grid=(N // _R,),
        in_specs=[row, row, pspec, wspec, wspec, wspec, bspec],
        out_specs=row,
        out_shape=jax.ShapeDtypeStruct((N, FEAT), _F32),
    )(cf0, cf1, p, ws0, ws1, ws2, bs)


def _sc_iter_body(a_hbm, bp_hbm, ii_hbm, out_hbm,
                  acc, ring, bufa0, bufb0, bufa1, bufb1, zbuf,
                  semg0, semg1, semi0, semi1):
    c = lax.axis_index("c")
    s = lax.axis_index("s")
    wid = s * NC + c

    bufs = ((bufa0, bufb0, semg0, semi0), (bufa1, bufb1, semg1, semi1))

    def idx_load(k2, sem):
        pltpu.async_copy(ii_hbm.at[wid, k2], ring.at[lax.rem(k2, 4)], sem)

    def issue(k, p):
        ba, bb, sg, _ = bufs[p]
        m = lax.rem(k, 4)
        pltpu.async_copy(a_hbm.at[ring.at[m, 0]], ba, sg)
        pltpu.async_copy(bp_hbm.at[ring.at[m, 1]], bb, sg)

    def process(k, p, pre, pre_idx):
        ba, bb, sg, si = bufs[p]
        m = lax.rem(k, 4)
        pltpu.make_async_copy(a_hbm.at[ring.at[m, 0]], ba, sg).wait()
        pltpu.make_async_copy(bp_hbm.at[ring.at[m, 1]], bb, sg).wait()

        def edge_body(e, _):
            for j in range(H // 16):
                sl = pl.ds(j * 16, 16)
                ba[e, sl] = jnp.maximum(ba[e, sl] + bb[e, sl], 0.0)
            return 0
        lax.fori_loop(0, CH, edge_body, 0)
        pltpu.sync_copy(ba, acc.at[ring.at[m, 0]], add=True)
        if pre:
            pltpu.make_async_copy(ii_hbm.at[wid, k + 2],
                                  ring.at[lax.rem(k + 2, 4)], si).wait()
            issue(k + 2, p)
        if pre_idx:
            idx_load(k + 4, si)

    pltpu.sync_copy(ii_hbm.at[wid, 0], ring.at[0])
    pltpu.sync_copy(ii_hbm.at[wid, 1], ring.at[1])
    issue(0, 0)
    issue(1, 1)
    idx_load(2, semi0)
    idx_load(3, semi1)

    def zero_body(e, _):
        for j in range(H // 16):
            zbuf[e, pl.ds(j * 16, 16)] = jnp.zeros((16,), _F32)
        return 0
    ZR = 40
    lax.fori_loop(0, ZR, zero_body, 0)
    for r in range(RPS // ZR):
        pltpu.sync_copy(zbuf.at[pl.ds(0, ZR)],
                        acc.at[pl.ds(s * RPS + r * ZR, ZR)])
    plsc.subcore_barrier()

    def pair_body(i, _):
        k0 = 2 * i
        process(k0, 0, True, True)
        process(k0 + 1, 1, True, True)
        return 0
    lax.fori_loop(0, (NCHUNK - 5) // 2, pair_body, 0)
    process(NCHUNK - 5, 0, True, True)
    process(NCHUNK - 4, 1, True, False)
    process(NCHUNK - 3, 0, True, False)
    process(NCHUNK - 2, 1, False, False)
    process(NCHUNK - 1, 0, False, False)

    plsc.subcore_barrier()

    pltpu.sync_copy(acc.at[pl.ds(s * RPS, RPS)],
                    out_hbm.at[c, pl.ds(s * RPS, RPS)])


_sc_iter = pl.kernel(
    _sc_iter_body,
    out_type=jax.ShapeDtypeStruct((NC, NP, H), _F32),
    mesh=plsc.VectorSubcoreMesh(core_axis_name="c", subcore_axis_name="s"),
    scratch_types=[
        pltpu.VMEM_SHARED((NP, H), _F32),
        pltpu.VMEM((4, 2, CH), jnp.int32),
        pltpu.VMEM((CH, H), _F32),
        pltpu.VMEM((CH, H), _F32),
        pltpu.VMEM((CH, H), _F32),
        pltpu.VMEM((CH, H), _F32),
        pltpu.VMEM((40, H), _F32),
        pltpu.SemaphoreType.DMA,
        pltpu.SemaphoreType.DMA,
        pltpu.SemaphoreType.DMA,
        pltpu.SemaphoreType.DMA,
    ],
)


def kernel(child_feats, edge_indices, edge_type_onehot, lengths,
           W1, b1, W2, b2, We0, be0, We1, be1, Ws, bs):
    x = child_feats[0]
    ef = edge_indices[0, :, 0].astype(jnp.int32)
    et = edge_indices[0, :, 1].astype(jnp.int32)
    ty = jnp.argmax(edge_type_onehot[0], axis=1).astype(jnp.int32)
    ebt = ty * N + et

    ii = jnp.stack([ef.reshape(NW, NCHUNK, CH),
                    ebt.reshape(NW, NCHUNK, CH)], axis=2)

    w1p = jnp.zeros((FEAT, H), _F32).at[:W1.shape[0]].set(W1)
    t0 = We0[2 * H:] + be0
    t1 = We1[2 * H:] + be1

    cf0, a0, bp0 = _tc_prep0(x, w1p, b1.reshape(1, H), W2, b2.reshape(1, H),
                             We0[:H], We0[H:2 * H], t0)
    p0 = _sc_iter(a0, bp0.reshape(ET * N, H), ii)
    cf1, a1, bp1 = _tc_prep1(p0, We1[:H], We1[H:2 * H], t1)
    p1 = _sc_iter(a1, bp1.reshape(ET * N, H), ii)
    return _tc_final(cf0, cf1, p1, Ws[:H], Ws[H:2 * H], Ws[2 * H:],
                     bs.reshape(1, FEAT))
